# baseline (speedup 1.0000x reference)
import functools

import jax
import jax.numpy as jnp
from jax import lax
from jax.experimental import pallas as pl
from jax.experimental.pallas import tpu as pltpu
from jax.experimental.pallas import tpu_sc as plsc

N_NODES = 10000
N_PAD = 10240
E = 320000
CH = 128
NC, NS = 2, 16
NW = NC * NS
EPT = 10240
E_PAD = NW * EPT
DEAD = N_PAD - 1
EK_C = 128
NCH = EPT // EK_C
RING = 2
NPT = N_PAD // NS

_MESH = plsc.VectorSubcoreMesh(core_axis_name="c", subcore_axis_name="s")


@functools.partial(
    pl.kernel,
    out_type=jax.ShapeDtypeStruct((NW * N_PAD,), jnp.float32),
    mesh=_MESH,
    scratch_types=[
        pltpu.VMEM((N_PAD,), jnp.float32),
        pltpu.VMEM((EPT,), jnp.int32),
    ],
    compiler_params=pltpu.CompilerParams(needs_layout_passes=False),
)
def _deg_kernel(dst_hbm, out_hbm, hist, idx):
    cid = lax.axis_index("c")
    sid = lax.axis_index("s")
    wid = sid * NC + cid

    zeros16 = jnp.zeros((16,), jnp.float32)
    ones16 = jnp.ones((16,), jnp.float32)

    def zero_body(i, carry):
        for u in range(5):
            hist[pl.ds((i * 5 + u) * 16, 16)] = zeros16
        return carry

    lax.fori_loop(0, N_PAD // 80, zero_body, 0)
    pltpu.sync_copy(dst_hbm.at[pl.ds(wid * EPT, EPT)], idx)

    def vec_body(j, carry):
        for u in range(5):
            iv = idx[pl.ds((j * 5 + u) * 16, 16)]
            plsc.addupdate_scatter(hist, [iv], ones16)
        return carry

    lax.fori_loop(0, EPT // 80, vec_body, 0)
    pltpu.sync_copy(hist, out_hbm.at[pl.ds(wid * N_PAD, N_PAD)])


def _scale_body(x_ref, p_ref, o_ref):
    ones = jnp.ones((NW, CH), jnp.float32)
    degb = lax.dot_general(
        p_ref[...], ones, (((0,), (0,)), ((), ())),
        preferred_element_type=jnp.float32,
    )[:N_NODES] + 1.0
    o_ref[...] = x_ref[...] * lax.rsqrt(degb)


def _scale_kernel(x, partials):
    return pl.pallas_call(
        _scale_body,
        out_shape=jax.ShapeDtypeStruct((N_NODES, CH), jnp.float32),
    )(x, partials)


@functools.partial(
    pl.kernel,
    out_type=jax.ShapeDtypeStruct((NC * N_PAD, CH), jnp.float32),
    mesh=_MESH,
    scratch_types=[
        pltpu.VMEM_SHARED((N_PAD, CH), jnp.float32),
        pltpu.VMEM((EPT,), jnp.int32),
    ]
    + [pltpu.VMEM((EK_C, CH), jnp.float32) for _ in range(RING)]
    + [pltpu.VMEM((EK_C,), jnp.int32) for _ in range(RING)]
    + [pltpu.SemaphoreType.DMA for _ in range(3 * RING)],
    compiler_params=pltpu.CompilerParams(needs_layout_passes=False),
)
def _agg_kernel(xp_hbm, src_hbm, dst_hbm, zero_hbm, out_hbm,
                acc, sidx, *bufs):
    rows = bufs[:RING]
    didx = bufs[RING : 2 * RING]
    semg = bufs[2 * RING : 3 * RING]
    sems = bufs[3 * RING : 4 * RING]
    semi = bufs[4 * RING :]
    cid = lax.axis_index("c")
    sid = lax.axis_index("s")
    wid = sid * NC + cid

    pltpu.sync_copy(src_hbm.at[pl.ds(wid * EPT, EPT)], sidx)
    pltpu.sync_copy(zero_hbm, acc.at[pl.ds(sid * NPT, NPT)])
    plsc.subcore_barrier()

    for b in range(RING):
        pltpu.async_copy(dst_hbm.at[pl.ds(wid * EPT + b * EK_C, EK_C)],
                         didx[b], semi[b])
        pltpu.async_copy(xp_hbm.at[sidx.at[pl.ds(b * EK_C, EK_C)]],
                         rows[b], semg[b])

    def group_body(g, carry):
        for b in range(RING):
            c = g * RING + b
            pltpu.make_async_copy(dst_hbm.at[pl.ds(0, EK_C)], didx[b],
                                  semi[b]).wait()
            pltpu.make_async_copy(xp_hbm.at[sidx.at[pl.ds(0, EK_C)]],
                                  rows[b], semg[b]).wait()
            pltpu.async_copy(rows[b], acc.at[didx[b]], sems[b],
                             add=True).wait()

            @pl.when(g < NCH // RING - 1)
            def _():
                base = wid * EPT + (c + RING) * EK_C
                pltpu.async_copy(dst_hbm.at[pl.ds(base, EK_C)], didx[b],
                                 semi[b])
                pltpu.async_copy(
                    xp_hbm.at[sidx.at[pl.ds((c + RING) * EK_C, EK_C)]],
                    rows[b], semg[b])

        return carry

    lax.fori_loop(0, NCH // RING, group_body, 0)
    plsc.subcore_barrier()

    pltpu.sync_copy(acc.at[pl.ds(sid * NPT, NPT)],
                    out_hbm.at[pl.ds(cid * N_PAD + sid * NPT, NPT)])


def _final_body(p_ref, x_ref, a0_ref, a1_ref, w_ref, b_ref, g_ref, bb_ref,
                o_ref):
    ones = jnp.ones((NW, CH), jnp.float32)
    degb = lax.dot_general(
        p_ref[...], ones, (((0,), (0,)), ((), ())),
        preferred_element_type=jnp.float32,
    )[:N_NODES] + 1.0
    agg = lax.rsqrt(degb) * (a0_ref[...] + a1_ref[...]) + x_ref[...] / degb
    h = jnp.dot(agg, w_ref[...], preferred_element_type=jnp.float32)
    h = jnp.maximum(h + b_ref[...], 0.0)
    m = jnp.mean(h, axis=0, keepdims=True)
    v = jnp.mean((h - m) * (h - m), axis=0, keepdims=True)
    hn = (h - m) * lax.rsqrt(v + 1e-5) * g_ref[...] + bb_ref[...]
    o_ref[...] = jnp.maximum(hn[:, : CH // 2], hn[:, CH // 2 :])


def _final_kernel(partials, x, acc0, acc1, Wr, br, gr, betar):
    return pl.pallas_call(
        _final_body,
        out_shape=jax.ShapeDtypeStruct((N_NODES, CH // 2), jnp.float32),
    )(partials, x, acc0, acc1, Wr, br, gr, betar)


def kernel(x, edge_index, W, b, bn_gamma, bn_beta):
    src = edge_index[0].astype(jnp.int32)
    dst = edge_index[1].astype(jnp.int32)
    npad = E_PAD - E
    src = jnp.concatenate([src, jnp.zeros((npad,), jnp.int32)])
    dst = jnp.concatenate([dst, jnp.full((npad,), DEAD, jnp.int32)])

    partials = _deg_kernel(dst).reshape(NW, N_PAD)
    xprime = _scale_kernel(x, partials)
    zeros = jnp.zeros((NPT, CH), jnp.float32)
    accs = _agg_kernel(xprime, src, dst, zeros)

    Wr = jnp.concatenate([W[:, 0::2], W[:, 1::2]], axis=1)
    br = jnp.concatenate([b[0::2], b[1::2]]).reshape(1, CH)
    gr = jnp.concatenate([bn_gamma[0::2], bn_gamma[1::2]]).reshape(1, CH)
    betar = jnp.concatenate([bn_beta[0::2], bn_beta[1::2]]).reshape(1, CH)

    return _final_kernel(partials, x, accs[:N_NODES],
                         accs[N_PAD : N_PAD + N_NODES], Wr, br, gr, betar)


# speedup vs baseline: 1.0006x; 1.0006x over previous
import functools

import jax
import jax.numpy as jnp
from jax import lax
from jax.experimental import pallas as pl
from jax.experimental.pallas import tpu as pltpu
from jax.experimental.pallas import tpu_sc as plsc

N_NODES = 10000
N_PAD = 10240
E = 320000
CH = 128
NC, NS = 2, 16
NW = NC * NS
EPT = 10240
E_PAD = NW * EPT
DEAD = N_PAD - 1
EK_C = 64
NCH = EPT // EK_C
RING = 4
NPT = N_PAD // NS

_MESH = plsc.VectorSubcoreMesh(core_axis_name="c", subcore_axis_name="s")


@functools.partial(
    pl.kernel,
    out_type=jax.ShapeDtypeStruct((NW * N_PAD,), jnp.float32),
    mesh=_MESH,
    scratch_types=[
        pltpu.VMEM((N_PAD,), jnp.float32),
        pltpu.VMEM((EPT,), jnp.int32),
    ],
    compiler_params=pltpu.CompilerParams(needs_layout_passes=False),
)
def _deg_kernel(dst_hbm, out_hbm, hist, idx):
    cid = lax.axis_index("c")
    sid = lax.axis_index("s")
    wid = sid * NC + cid

    zeros16 = jnp.zeros((16,), jnp.float32)
    ones16 = jnp.ones((16,), jnp.float32)

    def zero_body(i, carry):
        for u in range(5):
            hist[pl.ds((i * 5 + u) * 16, 16)] = zeros16
        return carry

    lax.fori_loop(0, N_PAD // 80, zero_body, 0)
    pltpu.sync_copy(dst_hbm.at[pl.ds(wid * EPT, EPT)], idx)

    def vec_body(j, carry):
        for u in range(5):
            iv = idx[pl.ds((j * 5 + u) * 16, 16)]
            plsc.addupdate_scatter(hist, [iv], ones16)
        return carry

    lax.fori_loop(0, EPT // 80, vec_body, 0)
    pltpu.sync_copy(hist, out_hbm.at[pl.ds(wid * N_PAD, N_PAD)])


def _scale_body(x_ref, p_ref, o_ref):
    ones = jnp.ones((NW, CH), jnp.float32)
    degb = lax.dot_general(
        p_ref[...], ones, (((0,), (0,)), ((), ())),
        preferred_element_type=jnp.float32,
    )[:N_NODES] + 1.0
    o_ref[...] = x_ref[...] * lax.rsqrt(degb)


def _scale_kernel(x, partials):
    return pl.pallas_call(
        _scale_body,
        out_shape=jax.ShapeDtypeStruct((N_NODES, CH), jnp.float32),
    )(x, partials)


@functools.partial(
    pl.kernel,
    out_type=jax.ShapeDtypeStruct((NC * N_PAD, CH), jnp.float32),
    mesh=_MESH,
    scratch_types=[
        pltpu.VMEM_SHARED((N_PAD, CH), jnp.float32),
        pltpu.VMEM((EPT,), jnp.int32),
    ]
    + [pltpu.VMEM((EK_C, CH), jnp.float32) for _ in range(RING)]
    + [pltpu.VMEM((EK_C,), jnp.int32) for _ in range(RING)]
    + [pltpu.SemaphoreType.DMA for _ in range(3 * RING)],
    compiler_params=pltpu.CompilerParams(needs_layout_passes=False),
)
def _agg_kernel(xp_hbm, src_hbm, dst_hbm, zero_hbm, out_hbm,
                acc, sidx, *bufs):
    rows = bufs[:RING]
    didx = bufs[RING : 2 * RING]
    semg = bufs[2 * RING : 3 * RING]
    sems = bufs[3 * RING : 4 * RING]
    semi = bufs[4 * RING :]
    cid = lax.axis_index("c")
    sid = lax.axis_index("s")
    wid = sid * NC + cid

    pltpu.sync_copy(src_hbm.at[pl.ds(wid * EPT, EPT)], sidx)
    pltpu.sync_copy(zero_hbm, acc.at[pl.ds(sid * NPT, NPT)])
    plsc.subcore_barrier()

    for b in range(RING):
        pltpu.async_copy(dst_hbm.at[pl.ds(wid * EPT + b * EK_C, EK_C)],
                         didx[b], semi[b])
        pltpu.async_copy(xp_hbm.at[sidx.at[pl.ds(b * EK_C, EK_C)]],
                         rows[b], semg[b])

    def group_body(g, carry):
        for b in range(RING):
            c = g * RING + b
            pltpu.make_async_copy(dst_hbm.at[pl.ds(0, EK_C)], didx[b],
                                  semi[b]).wait()
            pltpu.make_async_copy(xp_hbm.at[sidx.at[pl.ds(0, EK_C)]],
                                  rows[b], semg[b]).wait()
            pltpu.async_copy(rows[b], acc.at[didx[b]], sems[b],
                             add=True).wait()

            @pl.when(g < NCH // RING - 1)
            def _():
                base = wid * EPT + (c + RING) * EK_C
                pltpu.async_copy(dst_hbm.at[pl.ds(base, EK_C)], didx[b],
                                 semi[b])
                pltpu.async_copy(
                    xp_hbm.at[sidx.at[pl.ds((c + RING) * EK_C, EK_C)]],
                    rows[b], semg[b])

        return carry

    lax.fori_loop(0, NCH // RING, group_body, 0)
    plsc.subcore_barrier()

    pltpu.sync_copy(acc.at[pl.ds(sid * NPT, NPT)],
                    out_hbm.at[pl.ds(cid * N_PAD + sid * NPT, NPT)])


def _final_body(p_ref, x_ref, a0_ref, a1_ref, w_ref, b_ref, g_ref, bb_ref,
                o_ref):
    ones = jnp.ones((NW, CH), jnp.float32)
    degb = lax.dot_general(
        p_ref[...], ones, (((0,), (0,)), ((), ())),
        preferred_element_type=jnp.float32,
    )[:N_NODES] + 1.0
    agg = lax.rsqrt(degb) * (a0_ref[...] + a1_ref[...]) + x_ref[...] / degb
    h = jnp.dot(agg, w_ref[...], preferred_element_type=jnp.float32)
    h = jnp.maximum(h + b_ref[...], 0.0)
    m = jnp.mean(h, axis=0, keepdims=True)
    v = jnp.mean((h - m) * (h - m), axis=0, keepdims=True)
    hn = (h - m) * lax.rsqrt(v + 1e-5) * g_ref[...] + bb_ref[...]
    o_ref[...] = jnp.maximum(hn[:, : CH // 2], hn[:, CH // 2 :])


def _final_kernel(partials, x, acc0, acc1, Wr, br, gr, betar):
    return pl.pallas_call(
        _final_body,
        out_shape=jax.ShapeDtypeStruct((N_NODES, CH // 2), jnp.float32),
    )(partials, x, acc0, acc1, Wr, br, gr, betar)


def kernel(x, edge_index, W, b, bn_gamma, bn_beta):
    src = edge_index[0].astype(jnp.int32)
    dst = edge_index[1].astype(jnp.int32)
    npad = E_PAD - E
    src = jnp.concatenate([src, jnp.zeros((npad,), jnp.int32)])
    dst = jnp.concatenate([dst, jnp.full((npad,), DEAD, jnp.int32)])

    partials = _deg_kernel(dst).reshape(NW, N_PAD)
    xprime = _scale_kernel(x, partials)
    zeros = jnp.zeros((NPT, CH), jnp.float32)
    accs = _agg_kernel(xprime, src, dst, zeros)

    Wr = jnp.concatenate([W[:, 0::2], W[:, 1::2]], axis=1)
    br = jnp.concatenate([b[0::2], b[1::2]]).reshape(1, CH)
    gr = jnp.concatenate([bn_gamma[0::2], bn_gamma[1::2]]).reshape(1, CH)
    betar = jnp.concatenate([bn_beta[0::2], bn_beta[1::2]]).reshape(1, CH)

    return _final_kernel(partials, x, accs[:N_NODES],
                         accs[N_PAD : N_PAD + N_NODES], Wr, br, gr, betar)


# speedup vs baseline: 3.4518x; 3.4496x over previous
import functools

import jax
import jax.numpy as jnp
from jax import lax
from jax.experimental import pallas as pl
from jax.experimental.pallas import tpu as pltpu
from jax.experimental.pallas import tpu_sc as plsc

N_NODES = 10000
N_PAD = 10240
E = 320000
CH = 128
NC, NS = 2, 16
NW = NC * NS
EPT = E // NW
EK_C = 40
NCH = EPT // EK_C
RING = 5
NPT = N_PAD // NS

_MESH = plsc.VectorSubcoreMesh(core_axis_name="c", subcore_axis_name="s")


@functools.partial(
    pl.kernel,
    out_type=jax.ShapeDtypeStruct((NW * N_PAD,), jnp.float32),
    mesh=_MESH,
    scratch_types=[
        pltpu.VMEM((N_PAD,), jnp.float32),
        pltpu.VMEM((EPT,), jnp.int32),
    ],
    compiler_params=pltpu.CompilerParams(needs_layout_passes=False),
)
def _deg_kernel(dst_hbm, out_hbm, hist, idx):
    cid = lax.axis_index("c")
    sid = lax.axis_index("s")
    wid = sid * NC + cid

    zeros16 = jnp.zeros((16,), jnp.float32)
    ones16 = jnp.ones((16,), jnp.float32)

    def zero_body(i, carry):
        for u in range(5):
            hist[pl.ds((i * 5 + u) * 16, 16)] = zeros16
        return carry

    lax.fori_loop(0, N_PAD // 80, zero_body, 0)
    pltpu.sync_copy(dst_hbm.at[pl.ds(wid * EPT, EPT)], idx)

    def vec_body(j, carry):
        for u in range(5):
            iv = idx[pl.ds((j * 5 + u) * 16, 16)]
            plsc.addupdate_scatter(hist, [iv], ones16)
        return carry

    lax.fori_loop(0, EPT // 80, vec_body, 0)
    pltpu.sync_copy(hist, out_hbm.at[pl.ds(wid * N_PAD, N_PAD)])


def _scale_body(x_ref, p_ref, o_ref):
    ones = jnp.ones((NW, CH), jnp.float32)
    degb = lax.dot_general(
        p_ref[...], ones, (((0,), (0,)), ((), ())),
        preferred_element_type=jnp.float32,
    )[:N_NODES] + 1.0
    o_ref[...] = x_ref[...] * lax.rsqrt(degb)


def _scale_kernel(x, partials):
    return pl.pallas_call(
        _scale_body,
        out_shape=jax.ShapeDtypeStruct((N_NODES, CH), jnp.float32),
    )(x, partials)


@functools.partial(
    pl.kernel,
    out_type=jax.ShapeDtypeStruct((NC * N_PAD, CH), jnp.float32),
    mesh=_MESH,
    scratch_types=[
        pltpu.VMEM_SHARED((N_PAD, CH), jnp.float32),
        pltpu.VMEM((EPT,), jnp.int32),
    ]
    + [pltpu.VMEM((EK_C, CH), jnp.float32) for _ in range(RING)]
    + [pltpu.VMEM((EK_C,), jnp.int32) for _ in range(RING)]
    + [pltpu.SemaphoreType.DMA for _ in range(3 * RING)],
    compiler_params=pltpu.CompilerParams(needs_layout_passes=False),
)
def _agg_kernel(xp_hbm, src_hbm, dst_hbm, zero_hbm, out_hbm,
                acc, sidx, *bufs):
    rows = bufs[:RING]
    didx = bufs[RING : 2 * RING]
    semg = bufs[2 * RING : 3 * RING]
    sems = bufs[3 * RING : 4 * RING]
    semi = bufs[4 * RING :]
    cid = lax.axis_index("c")
    sid = lax.axis_index("s")
    wid = sid * NC + cid

    pltpu.sync_copy(src_hbm.at[pl.ds(wid * EPT, EPT)], sidx)
    pltpu.sync_copy(zero_hbm, acc.at[pl.ds(sid * NPT, NPT)])
    plsc.subcore_barrier()

    for b in range(RING):
        pltpu.async_copy(dst_hbm.at[pl.ds(wid * EPT + b * EK_C, EK_C)],
                         didx[b], semi[b])
        pltpu.async_copy(xp_hbm.at[sidx.at[pl.ds(b * EK_C, EK_C)]],
                         rows[b], semg[b])

    def group_body(g, carry):
        for b in range(RING):
            c = g * RING + b
            pltpu.make_async_copy(dst_hbm.at[pl.ds(0, EK_C)], didx[b],
                                  semi[b]).wait()
            pltpu.make_async_copy(xp_hbm.at[sidx.at[pl.ds(0, EK_C)]],
                                  rows[b], semg[b]).wait()
            pltpu.async_copy(rows[b], acc.at[didx[b]], sems[b],
                             add=True).wait()

            @pl.when(g < NCH // RING - 1)
            def _():
                base = wid * EPT + (c + RING) * EK_C
                pltpu.async_copy(dst_hbm.at[pl.ds(base, EK_C)], didx[b],
                                 semi[b])
                pltpu.async_copy(
                    xp_hbm.at[sidx.at[pl.ds((c + RING) * EK_C, EK_C)]],
                    rows[b], semg[b])

        return carry

    lax.fori_loop(0, NCH // RING, group_body, 0)
    plsc.subcore_barrier()

    pltpu.sync_copy(acc.at[pl.ds(sid * NPT, NPT)],
                    out_hbm.at[pl.ds(cid * N_PAD + sid * NPT, NPT)])


def _final_body(p_ref, x_ref, a_ref, w_ref, b_ref, g_ref, bb_ref,
                o_ref):
    ones = jnp.ones((NW, CH), jnp.float32)
    degb = lax.dot_general(
        p_ref[...], ones, (((0,), (0,)), ((), ())),
        preferred_element_type=jnp.float32,
    )[:N_NODES] + 1.0
    asum = (a_ref[0 : N_NODES, :]
            + a_ref[N_PAD : N_PAD + N_NODES, :])
    agg = lax.rsqrt(degb) * asum + x_ref[...] / degb
    h = jnp.dot(agg, w_ref[...], preferred_element_type=jnp.float32)
    h = jnp.maximum(h + b_ref[...], 0.0)
    m = jnp.mean(h, axis=0, keepdims=True)
    v = jnp.mean((h - m) * (h - m), axis=0, keepdims=True)
    hn = (h - m) * lax.rsqrt(v + 1e-5) * g_ref[...] + bb_ref[...]
    o_ref[...] = jnp.maximum(hn[:, : CH // 2], hn[:, CH // 2 :])


def _final_kernel(partials, x, accs, Wr, br, gr, betar):
    return pl.pallas_call(
        _final_body,
        out_shape=jax.ShapeDtypeStruct((N_NODES, CH // 2), jnp.float32),
    )(partials, x, accs, Wr, br, gr, betar)


def kernel(x, edge_index, W, b, bn_gamma, bn_beta):
    src = edge_index[0].astype(jnp.int32)
    dst = edge_index[1].astype(jnp.int32)

    partials = _deg_kernel(dst).reshape(NW, N_PAD)
    xprime = _scale_kernel(x, partials)
    zeros = jnp.zeros((NPT, CH), jnp.float32)
    accs = _agg_kernel(xprime, src, dst, zeros)

    Wr = jnp.concatenate([W[:, 0::2], W[:, 1::2]], axis=1)
    br = jnp.concatenate([b[0::2], b[1::2]]).reshape(1, CH)
    gr = jnp.concatenate([bn_gamma[0::2], bn_gamma[1::2]]).reshape(1, CH)
    betar = jnp.concatenate([bn_beta[0::2], bn_beta[1::2]]).reshape(1, CH)

    return _final_kernel(partials, x, accs, Wr, br, gr, betar)
